# Initial kernel scaffold; baseline (speedup 1.0000x reference)
import functools

import jax
import jax.numpy as jnp
from jax import lax
from jax.experimental import pallas as pl
from jax.experimental.pallas import tpu as pltpu
from jax.experimental.pallas import tpu_sc as plsc

BATCH = 16384
TIMESTEPS = 200
EMBED_DIM = 32

NC = 2
NS = 16
NW = NC * NS

CH = 128
GRP = 8
TOTAL = BATCH * TIMESTEPS
PER_W = TOTAL // NW
NCH = PER_W // CH
NGRP = NCH // GRP


def _make_kernel():
    mesh = plsc.VectorSubcoreMesh(core_axis_name="c", subcore_axis_name="s")

    @functools.partial(
        pl.kernel,
        mesh=mesh,
        out_type=jax.ShapeDtypeStruct((NW, NCH, CH, EMBED_DIM), jnp.float32),
        scratch_types=[
            pltpu.VMEM((GRP, CH), jnp.int32),
            pltpu.VMEM((GRP, CH, EMBED_DIM), jnp.float32),
            pltpu.SemaphoreType.DMA,
        ],
    )
    def emb_kernel(items_hbm, table_hbm, out_hbm, idx_v, rows_v, sem):
        wid = lax.axis_index("s") * NC + lax.axis_index("c")

        def body(g, carry):
            pltpu.sync_copy(items_hbm.at[wid, pl.ds(g * GRP, GRP)], idx_v)
            copies = [
                pltpu.async_copy(table_hbm.at[idx_v.at[j]], rows_v.at[j], sem)
                for j in range(GRP)
            ]
            for c in copies:
                c.wait()
            pltpu.sync_copy(rows_v, out_hbm.at[wid, pl.ds(g * GRP, GRP)])
            return carry

        lax.fori_loop(0, NGRP, body, 0)

    return emb_kernel


_EMB = _make_kernel()


def kernel(items, table):
    idx = items.reshape(NW, NCH, CH).astype(jnp.int32)
    out = _EMB(idx, table)
    return out.reshape(BATCH, TIMESTEPS, EMBED_DIM)


# speedup vs baseline: 6.1356x; 6.1356x over previous
import functools

import jax
import jax.numpy as jnp
from jax import lax
from jax.experimental import pallas as pl
from jax.experimental.pallas import tpu as pltpu
from jax.experimental.pallas import tpu_sc as plsc

BATCH = 16384
TIMESTEPS = 200
EMBED_DIM = 32

NC = 2
NS = 16
NW = NC * NS

CH = 128
GRP = 8
TOTAL = BATCH * TIMESTEPS
PER_W = TOTAL // NW
NCH = PER_W // CH
NGRP = NCH // GRP


def _make_kernel():
    mesh = plsc.VectorSubcoreMesh(core_axis_name="c", subcore_axis_name="s")

    @functools.partial(
        pl.kernel,
        mesh=mesh,
        out_type=jax.ShapeDtypeStruct((NW, NCH, CH, EMBED_DIM), jnp.float32),
        scratch_types=[
            pltpu.VMEM((GRP, CH), jnp.int32),
            pltpu.VMEM((GRP, CH, EMBED_DIM), jnp.float32),
            pltpu.SemaphoreType.DMA,
        ],
        compiler_params=pltpu.CompilerParams(use_tc_tiling_on_sc=False),
    )
    def emb_kernel(items_hbm, table_hbm, out_hbm, idx_v, rows_v, sem):
        wid = lax.axis_index("s") * NC + lax.axis_index("c")

        def body(g, carry):
            pltpu.sync_copy(items_hbm.at[wid, pl.ds(g * GRP, GRP)], idx_v)
            copies = [
                pltpu.async_copy(table_hbm.at[idx_v.at[j]], rows_v.at[j], sem)
                for j in range(GRP)
            ]
            for c in copies:
                c.wait()
            pltpu.sync_copy(rows_v, out_hbm.at[wid, pl.ds(g * GRP, GRP)])
            return carry

        lax.fori_loop(0, NGRP, body, 0)

    return emb_kernel


_EMB = _make_kernel()


def kernel(items, table):
    idx = items.reshape(NW, NCH, CH).astype(jnp.int32)
    out = _EMB(idx, table)
    return out.reshape(BATCH, TIMESTEPS, EMBED_DIM)


# speedup vs baseline: 6.4914x; 1.0580x over previous
import functools

import jax
import jax.numpy as jnp
from jax import lax
from jax.experimental import pallas as pl
from jax.experimental.pallas import tpu as pltpu
from jax.experimental.pallas import tpu_sc as plsc

BATCH = 16384
TIMESTEPS = 200
EMBED_DIM = 32

NC = 2
NS = 16
NW = NC * NS

CH = 128
GRP = 5
NBUF = 4
TOTAL = BATCH * TIMESTEPS
PER_W = TOTAL // NW
NCH = PER_W // CH
NGRP = NCH // GRP
NOUT = NGRP // NBUF


def _make_kernel():
    mesh = plsc.VectorSubcoreMesh(core_axis_name="c", subcore_axis_name="s")

    @functools.partial(
        pl.kernel,
        mesh=mesh,
        out_type=jax.ShapeDtypeStruct((NW, NCH, CH, EMBED_DIM), jnp.float32),
        scratch_types=[
            pltpu.VMEM((NBUF, GRP, CH), jnp.int32),
            pltpu.VMEM((NBUF, GRP, CH, EMBED_DIM), jnp.float32),
        ] + [pltpu.SemaphoreType.DMA] * (2 * NBUF),
        compiler_params=pltpu.CompilerParams(use_tc_tiling_on_sc=False),
    )
    def emb_kernel(items_hbm, table_hbm, out_hbm, idx_v, rows_v, *sems):
        gsem = sems[:NBUF]
        osem = sems[NBUF:]
        wid = lax.axis_index("s") * NC + lax.axis_index("c")

        def stage_and_fire(g, b):
            pltpu.sync_copy(items_hbm.at[wid, pl.ds(g * GRP, GRP)], idx_v.at[b])
            for j in range(GRP):
                pltpu.async_copy(
                    table_hbm.at[idx_v.at[b, j]], rows_v.at[b, j], gsem[b])

        def wait_gathers(b):
            for j in range(GRP):
                pltpu.make_async_copy(
                    table_hbm.at[idx_v.at[b, j]], rows_v.at[b, j], gsem[b]
                ).wait()

        def fire_store(g, b):
            pltpu.async_copy(
                rows_v.at[b], out_hbm.at[wid, pl.ds(g * GRP, GRP)], osem[b])

        def wait_store(g, b):
            pltpu.make_async_copy(
                rows_v.at[b], out_hbm.at[wid, pl.ds(g * GRP, GRP)], osem[b]
            ).wait()

        for g0 in range(NBUF):
            stage_and_fire(g0, g0)
        for g0 in range(2):
            wait_gathers(g0)
            fire_store(g0, g0)

        def body(outer, carry):
            for b in range(NBUF):
                g = outer * NBUF + b
                b2 = (b + 2) % NBUF
                wait_gathers(b2)
                fire_store(g - 2, b2)
                wait_store(g - NBUF, b)
                stage_and_fire(g, b)
            return carry

        lax.fori_loop(1, NOUT, body, 0)

        for g0 in (NGRP - 2, NGRP - 1):
            b = g0 % NBUF
            wait_gathers(b)
            fire_store(g0, b)
        for b in range(NBUF):
            g_last = NGRP - NBUF + b
            wait_store(g_last, b)

    return emb_kernel


_EMB = _make_kernel()


def kernel(items, table):
    idx = items.reshape(NW, NCH, CH).astype(jnp.int32)
    out = _EMB(idx, table)
    return out.reshape(BATCH, TIMESTEPS, EMBED_DIM)
